# baseline (device time: 85273 ns/iter reference)
import functools

import jax
import jax.numpy as jnp
from jax import lax
from jax.experimental import pallas as pl
from jax.experimental.pallas import tpu as pltpu

N_DEV = 4


def kernel(x, w_mat, scale_x, scale_w):
    m_per, k = x.shape
    _, n_per = w_mat.shape
    half = m_per // 2

    def body(x_hbm, w_hbm, sx_ref, sw_ref, out_hbm,
             x_vmem, w_vmem, buf_l, buf_r, buf_l2, buf_r2, stage,
             local_sems, out_sems, send_sems, recv_sems):
        my = lax.axis_index("i")
        left = lax.rem(my + (N_DEV - 1), N_DEV)
        right = lax.rem(my + 1, N_DEV)
        opp = lax.rem(my + 2, N_DEV)

        cp_x = pltpu.make_async_copy(x_hbm, x_vmem, local_sems.at[0])
        cp_w = pltpu.make_async_copy(w_hbm, w_vmem, local_sems.at[1])
        cp_x.start()
        cp_w.start()

        barrier_sem = pltpu.get_barrier_semaphore()
        for nbr in (left, right):
            pl.semaphore_signal(
                barrier_sem, inc=1,
                device_id=(nbr,), device_id_type=pl.DeviceIdType.MESH,
            )
        pl.semaphore_wait(barrier_sem, 2)

        send_r1 = pltpu.make_async_remote_copy(
            src_ref=x_hbm.at[pl.ds(0, half)],
            dst_ref=buf_l.at[pl.ds(0, half)],
            send_sem=send_sems.at[0], recv_sem=recv_sems.at[0],
            device_id=(right,), device_id_type=pl.DeviceIdType.MESH,
        )
        send_l1 = pltpu.make_async_remote_copy(
            src_ref=x_hbm.at[pl.ds(half, half)],
            dst_ref=buf_r.at[pl.ds(half, half)],
            send_sem=send_sems.at[1], recv_sem=recv_sems.at[1],
            device_id=(left,), device_id_type=pl.DeviceIdType.MESH,
        )
        send_r2 = pltpu.make_async_remote_copy(
            src_ref=x_hbm.at[pl.ds(half, half)],
            dst_ref=buf_l.at[pl.ds(half, half)],
            send_sem=send_sems.at[2], recv_sem=recv_sems.at[2],
            device_id=(right,), device_id_type=pl.DeviceIdType.MESH,
        )
        send_l2 = pltpu.make_async_remote_copy(
            src_ref=x_hbm.at[pl.ds(0, half)],
            dst_ref=buf_r.at[pl.ds(0, half)],
            send_sem=send_sems.at[3], recv_sem=recv_sems.at[3],
            device_id=(left,), device_id_type=pl.DeviceIdType.MESH,
        )
        send_r1.start()
        send_l1.start()
        send_r2.start()
        send_l2.start()

        scale = sx_ref[0] * sw_ref[0]

        def emit_block(slot, origin_row, rows, value):
            stage[slot, pl.ds(0, rows), :] = value
            cp = pltpu.make_async_copy(
                stage.at[slot, pl.ds(0, rows)],
                out_hbm.at[pl.ds(origin_row, rows)],
                out_sems.at[slot],
            )
            cp.start()
            return cp

        cp_x.wait()
        cp_w.wait()
        acc = jnp.dot(x_vmem[...], w_vmem[...],
                      preferred_element_type=jnp.int32)
        st0 = emit_block(0, my * m_per, m_per,
                         acc.astype(jnp.float32) * scale)

        send_r1.wait_recv()
        fwd_r = pltpu.make_async_remote_copy(
            src_ref=buf_l.at[pl.ds(0, half)], dst_ref=buf_l2,
            send_sem=send_sems.at[4], recv_sem=recv_sems.at[4],
            device_id=(right,), device_id_type=pl.DeviceIdType.MESH,
        )
        fwd_r.start()
        send_l1.wait_recv()
        fwd_l = pltpu.make_async_remote_copy(
            src_ref=buf_r.at[pl.ds(half, half)], dst_ref=buf_r2,
            send_sem=send_sems.at[5], recv_sem=recv_sems.at[5],
            device_id=(left,), device_id_type=pl.DeviceIdType.MESH,
        )
        fwd_l.start()

        send_r2.wait_recv()
        acc = jnp.dot(buf_l[...], w_vmem[...],
                      preferred_element_type=jnp.int32)
        st1 = emit_block(1, left * m_per, m_per,
                         acc.astype(jnp.float32) * scale)
        send_l2.wait_recv()
        acc = jnp.dot(buf_r[...], w_vmem[...],
                      preferred_element_type=jnp.int32)
        st2 = emit_block(2, right * m_per, m_per,
                         acc.astype(jnp.float32) * scale)

        fwd_r.wait_recv()
        acc = jnp.dot(buf_l2[...], w_vmem[...],
                      preferred_element_type=jnp.int32)
        st3 = emit_block(3, opp * m_per, half,
                         acc.astype(jnp.float32) * scale)
        fwd_l.wait_recv()
        acc = jnp.dot(buf_r2[...], w_vmem[...],
                      preferred_element_type=jnp.int32)
        st4 = emit_block(4, opp * m_per + half, half,
                         acc.astype(jnp.float32) * scale)

        st0.wait()
        st1.wait()
        st2.wait()
        st3.wait()
        st4.wait()
        send_r1.wait_send()
        send_l1.wait_send()
        send_r2.wait_send()
        send_l2.wait_send()
        fwd_r.wait_send()
        fwd_l.wait_send()

        @functools.partial(
            pl.run_scoped, second_barrier=pltpu.SemaphoreType.REGULAR
        )
        def _(second_barrier):
            for nbr in (left, right):
                pl.semaphore_signal(
                    second_barrier, inc=1,
                    device_id=(nbr,), device_id_type=pl.DeviceIdType.MESH,
                )
            pl.semaphore_wait(second_barrier, 2)

    return pl.pallas_call(
        body,
        out_shape=jax.ShapeDtypeStruct((N_DEV * m_per, n_per), jnp.float32),
        in_specs=[
            pl.BlockSpec(memory_space=pl.ANY),
            pl.BlockSpec(memory_space=pl.ANY),
            pl.BlockSpec(memory_space=pltpu.VMEM),
            pl.BlockSpec(memory_space=pltpu.VMEM),
        ],
        out_specs=pl.BlockSpec(memory_space=pl.ANY),
        scratch_shapes=[
            pltpu.VMEM((m_per, k), x.dtype),
            pltpu.VMEM((k, n_per), w_mat.dtype),
            pltpu.VMEM((m_per, k), x.dtype),
            pltpu.VMEM((m_per, k), x.dtype),
            pltpu.VMEM((half, k), x.dtype),
            pltpu.VMEM((half, k), x.dtype),
            pltpu.VMEM((5, m_per, n_per), jnp.float32),
            pltpu.SemaphoreType.DMA((2,)),
            pltpu.SemaphoreType.DMA((5,)),
            pltpu.SemaphoreType.DMA((6,)),
            pltpu.SemaphoreType.DMA((6,)),
        ],
        compiler_params=pltpu.CompilerParams(collective_id=0),
    )(x, w_mat, scale_x, scale_w)


# device time: 82938 ns/iter; 1.0282x vs baseline; 1.0282x over previous
import functools

import jax
import jax.numpy as jnp
from jax import lax
from jax.experimental import pallas as pl
from jax.experimental.pallas import tpu as pltpu

N_DEV = 4


def kernel(x, w_mat, scale_x, scale_w):
    m_per, k = x.shape
    _, n_per = w_mat.shape
    half = m_per // 2
    quart = m_per // 4

    def body(x_hbm, w_hbm, sx_ref, sw_ref, out_hbm,
             x_vmem, w_vmem, buf_l, buf_r, buf_l2, buf_r2, stage,
             local_sems, out_sems, send_sems, recv_sems):
        my = lax.axis_index("i")
        left = lax.rem(my + (N_DEV - 1), N_DEV)
        right = lax.rem(my + 1, N_DEV)
        opp = lax.rem(my + 2, N_DEV)

        cp_x = pltpu.make_async_copy(x_hbm, x_vmem, local_sems.at[0])
        cp_w = pltpu.make_async_copy(w_hbm, w_vmem, local_sems.at[1])
        cp_x.start()
        cp_w.start()

        barrier_sem = pltpu.get_barrier_semaphore()
        for nbr in (left, right):
            pl.semaphore_signal(
                barrier_sem, inc=1,
                device_id=(nbr,), device_id_type=pl.DeviceIdType.MESH,
            )
        pl.semaphore_wait(barrier_sem, 2)

        send_r1 = pltpu.make_async_remote_copy(
            src_ref=x_hbm.at[pl.ds(0, half)],
            dst_ref=buf_l.at[pl.ds(0, half)],
            send_sem=send_sems.at[0], recv_sem=recv_sems.at[0],
            device_id=(right,), device_id_type=pl.DeviceIdType.MESH,
        )
        send_l1 = pltpu.make_async_remote_copy(
            src_ref=x_hbm.at[pl.ds(half, half)],
            dst_ref=buf_r.at[pl.ds(half, half)],
            send_sem=send_sems.at[1], recv_sem=recv_sems.at[1],
            device_id=(left,), device_id_type=pl.DeviceIdType.MESH,
        )
        send_r2 = pltpu.make_async_remote_copy(
            src_ref=x_hbm.at[pl.ds(half, half)],
            dst_ref=buf_l.at[pl.ds(half, half)],
            send_sem=send_sems.at[2], recv_sem=recv_sems.at[2],
            device_id=(right,), device_id_type=pl.DeviceIdType.MESH,
        )
        send_l2 = pltpu.make_async_remote_copy(
            src_ref=x_hbm.at[pl.ds(0, half)],
            dst_ref=buf_r.at[pl.ds(0, half)],
            send_sem=send_sems.at[3], recv_sem=recv_sems.at[3],
            device_id=(left,), device_id_type=pl.DeviceIdType.MESH,
        )
        send_r1.start()
        send_l1.start()
        send_r2.start()
        send_l2.start()

        scale = sx_ref[0] * sw_ref[0]

        def emit(slot, stage_row, out_row, rows, value, sem_idx):
            stage[slot, pl.ds(stage_row, rows), :] = value
            cp = pltpu.make_async_copy(
                stage.at[slot, pl.ds(stage_row, rows)],
                out_hbm.at[pl.ds(out_row, rows)],
                out_sems.at[sem_idx],
            )
            cp.start()
            return cp

        cp_x.wait()
        cp_w.wait()
        acc = jnp.dot(x_vmem[...], w_vmem[...],
                      preferred_element_type=jnp.int32)
        st0 = emit(0, 0, my * m_per, m_per,
                   acc.astype(jnp.float32) * scale, 0)

        send_r1.wait_recv()
        fwd_r_a = pltpu.make_async_remote_copy(
            src_ref=buf_l.at[pl.ds(0, quart)],
            dst_ref=buf_l2.at[pl.ds(0, quart)],
            send_sem=send_sems.at[4], recv_sem=recv_sems.at[4],
            device_id=(right,), device_id_type=pl.DeviceIdType.MESH,
        )
        fwd_r_b = pltpu.make_async_remote_copy(
            src_ref=buf_l.at[pl.ds(quart, quart)],
            dst_ref=buf_l2.at[pl.ds(quart, quart)],
            send_sem=send_sems.at[5], recv_sem=recv_sems.at[5],
            device_id=(right,), device_id_type=pl.DeviceIdType.MESH,
        )
        fwd_r_a.start()
        fwd_r_b.start()
        send_l1.wait_recv()
        fwd_l_a = pltpu.make_async_remote_copy(
            src_ref=buf_r.at[pl.ds(half, quart)],
            dst_ref=buf_r2.at[pl.ds(0, quart)],
            send_sem=send_sems.at[6], recv_sem=recv_sems.at[6],
            device_id=(left,), device_id_type=pl.DeviceIdType.MESH,
        )
        fwd_l_b = pltpu.make_async_remote_copy(
            src_ref=buf_r.at[pl.ds(half + quart, quart)],
            dst_ref=buf_r2.at[pl.ds(quart, quart)],
            send_sem=send_sems.at[7], recv_sem=recv_sems.at[7],
            device_id=(left,), device_id_type=pl.DeviceIdType.MESH,
        )
        fwd_l_a.start()
        fwd_l_b.start()

        send_r2.wait_recv()
        acc = jnp.dot(buf_l[...], w_vmem[...],
                      preferred_element_type=jnp.int32)
        st1 = emit(1, 0, left * m_per, m_per,
                   acc.astype(jnp.float32) * scale, 1)
        send_l2.wait_recv()
        acc = jnp.dot(buf_r[...], w_vmem[...],
                      preferred_element_type=jnp.int32)
        st2 = emit(2, 0, right * m_per, m_per,
                   acc.astype(jnp.float32) * scale, 2)

        fwd_r_a.wait_recv()
        acc = jnp.dot(buf_l2[pl.ds(0, quart), :], w_vmem[...],
                      preferred_element_type=jnp.int32)
        st3 = emit(3, 0, opp * m_per, quart,
                   acc.astype(jnp.float32) * scale, 3)
        fwd_l_a.wait_recv()
        acc = jnp.dot(buf_r2[pl.ds(0, quart), :], w_vmem[...],
                      preferred_element_type=jnp.int32)
        st4 = emit(4, 0, opp * m_per + half, quart,
                   acc.astype(jnp.float32) * scale, 4)
        fwd_r_b.wait_recv()
        acc = jnp.dot(buf_l2[pl.ds(quart, quart), :], w_vmem[...],
                      preferred_element_type=jnp.int32)
        st5 = emit(3, quart, opp * m_per + quart, quart,
                   acc.astype(jnp.float32) * scale, 5)
        fwd_l_b.wait_recv()
        acc = jnp.dot(buf_r2[pl.ds(quart, quart), :], w_vmem[...],
                      preferred_element_type=jnp.int32)
        st6 = emit(4, quart, opp * m_per + half + quart, quart,
                   acc.astype(jnp.float32) * scale, 6)

        for cp in (st0, st1, st2, st3, st4, st5, st6):
            cp.wait()
        for s in (send_r1, send_l1, send_r2, send_l2,
                  fwd_r_a, fwd_r_b, fwd_l_a, fwd_l_b):
            s.wait_send()

        @functools.partial(
            pl.run_scoped, second_barrier=pltpu.SemaphoreType.REGULAR
        )
        def _(second_barrier):
            for nbr in (left, right):
                pl.semaphore_signal(
                    second_barrier, inc=1,
                    device_id=(nbr,), device_id_type=pl.DeviceIdType.MESH,
                )
            pl.semaphore_wait(second_barrier, 2)

    return pl.pallas_call(
        body,
        out_shape=jax.ShapeDtypeStruct((N_DEV * m_per, n_per), jnp.float32),
        in_specs=[
            pl.BlockSpec(memory_space=pl.ANY),
            pl.BlockSpec(memory_space=pl.ANY),
            pl.BlockSpec(memory_space=pltpu.VMEM),
            pl.BlockSpec(memory_space=pltpu.VMEM),
        ],
        out_specs=pl.BlockSpec(memory_space=pl.ANY),
        scratch_shapes=[
            pltpu.VMEM((m_per, k), x.dtype),
            pltpu.VMEM((k, n_per), w_mat.dtype),
            pltpu.VMEM((m_per, k), x.dtype),
            pltpu.VMEM((m_per, k), x.dtype),
            pltpu.VMEM((half, k), x.dtype),
            pltpu.VMEM((half, k), x.dtype),
            pltpu.VMEM((5, m_per, n_per), jnp.float32),
            pltpu.SemaphoreType.DMA((2,)),
            pltpu.SemaphoreType.DMA((7,)),
            pltpu.SemaphoreType.DMA((8,)),
            pltpu.SemaphoreType.DMA((8,)),
        ],
        compiler_params=pltpu.CompilerParams(collective_id=0),
    )(x, w_mat, scale_x, scale_w)


# device time: 81572 ns/iter; 1.0454x vs baseline; 1.0167x over previous
import functools

import jax
import jax.numpy as jnp
from jax import lax
from jax.experimental import pallas as pl
from jax.experimental.pallas import tpu as pltpu

N_DEV = 4


def kernel(x, w_mat, scale_x, scale_w):
    m_per, k = x.shape
    _, n_per = w_mat.shape
    half = m_per // 2
    quart = m_per // 4

    def body(x_hbm, w_hbm, sx_ref, sw_ref, out_hbm,
             x_vmem, w_vmem, buf_l, buf_r, buf_l2, buf_r2, stage,
             local_sems, out_sems, send_sems, recv_sems):
        my = lax.axis_index("i")
        left = lax.rem(my + (N_DEV - 1), N_DEV)
        right = lax.rem(my + 1, N_DEV)
        opp = lax.rem(my + 2, N_DEV)

        cp_x = pltpu.make_async_copy(x_hbm, x_vmem, local_sems.at[0])
        cp_w = pltpu.make_async_copy(w_hbm, w_vmem, local_sems.at[1])
        cp_x.start()
        cp_w.start()

        barrier_sem = pltpu.get_barrier_semaphore()
        for nbr in (left, right):
            pl.semaphore_signal(
                barrier_sem, inc=1,
                device_id=(nbr,), device_id_type=pl.DeviceIdType.MESH,
            )
        pl.semaphore_wait(barrier_sem, 2)

        send_r1 = pltpu.make_async_remote_copy(
            src_ref=x_hbm.at[pl.ds(0, half)],
            dst_ref=buf_l.at[pl.ds(0, half)],
            send_sem=send_sems.at[0], recv_sem=recv_sems.at[0],
            device_id=(right,), device_id_type=pl.DeviceIdType.MESH,
        )
        send_l1 = pltpu.make_async_remote_copy(
            src_ref=x_hbm.at[pl.ds(half, half)],
            dst_ref=buf_r.at[pl.ds(half, half)],
            send_sem=send_sems.at[1], recv_sem=recv_sems.at[1],
            device_id=(left,), device_id_type=pl.DeviceIdType.MESH,
        )
        send_r2 = pltpu.make_async_remote_copy(
            src_ref=x_hbm.at[pl.ds(half, half)],
            dst_ref=buf_l.at[pl.ds(half, half)],
            send_sem=send_sems.at[2], recv_sem=recv_sems.at[2],
            device_id=(right,), device_id_type=pl.DeviceIdType.MESH,
        )
        send_l2 = pltpu.make_async_remote_copy(
            src_ref=x_hbm.at[pl.ds(0, half)],
            dst_ref=buf_r.at[pl.ds(0, half)],
            send_sem=send_sems.at[3], recv_sem=recv_sems.at[3],
            device_id=(left,), device_id_type=pl.DeviceIdType.MESH,
        )
        send_r1.start()
        send_l1.start()
        send_r2.start()
        send_l2.start()

        scale = sx_ref[0] * sw_ref[0]

        def emit(slot, stage_row, out_row, rows, value, sem_idx):
            stage[slot, pl.ds(stage_row, rows), :] = value
            cp = pltpu.make_async_copy(
                stage.at[slot, pl.ds(stage_row, rows)],
                out_hbm.at[pl.ds(out_row, rows)],
                out_sems.at[sem_idx],
            )
            cp.start()
            return cp

        cp_x.wait()
        cp_w.wait()
        acc = jnp.dot(x_vmem[...], w_vmem[...],
                      preferred_element_type=jnp.int32)
        st0 = emit(0, 0, my * m_per, m_per,
                   acc.astype(jnp.float32) * scale, 0)

        eighth = m_per // 8
        r_pieces = [(0, quart), (quart, eighth), (quart + eighth, eighth)]
        send_r1.wait_recv()
        fwd_r = []
        for j, (off, rows) in enumerate(r_pieces):
            c = pltpu.make_async_remote_copy(
                src_ref=buf_l.at[pl.ds(off, rows)],
                dst_ref=buf_l2.at[pl.ds(off, rows)],
                send_sem=send_sems.at[4 + j], recv_sem=recv_sems.at[4 + j],
                device_id=(right,), device_id_type=pl.DeviceIdType.MESH,
            )
            c.start()
            fwd_r.append(c)
        send_l1.wait_recv()
        fwd_l = []
        for j, (off, rows) in enumerate(r_pieces):
            c = pltpu.make_async_remote_copy(
                src_ref=buf_r.at[pl.ds(half + off, rows)],
                dst_ref=buf_r2.at[pl.ds(off, rows)],
                send_sem=send_sems.at[7 + j], recv_sem=recv_sems.at[7 + j],
                device_id=(left,), device_id_type=pl.DeviceIdType.MESH,
            )
            c.start()
            fwd_l.append(c)

        send_r2.wait_recv()
        acc = jnp.dot(buf_l[...], w_vmem[...],
                      preferred_element_type=jnp.int32)
        st1 = emit(1, 0, left * m_per, m_per,
                   acc.astype(jnp.float32) * scale, 1)
        send_l2.wait_recv()
        acc = jnp.dot(buf_r[...], w_vmem[...],
                      preferred_element_type=jnp.int32)
        st2 = emit(2, 0, right * m_per, m_per,
                   acc.astype(jnp.float32) * scale, 2)

        outs = [st0, st1, st2]

        @functools.partial(
            pl.run_scoped, second_barrier=pltpu.SemaphoreType.REGULAR
        )
        def _(second_barrier):
            for j, (off, rows) in enumerate(r_pieces):
                last = j == len(r_pieces) - 1
                fwd_r[j].wait_recv()
                if last:
                    fwd_l[j].wait_recv()
                    for nbr in (left, right):
                        pl.semaphore_signal(
                            second_barrier, inc=1,
                            device_id=(nbr,),
                            device_id_type=pl.DeviceIdType.MESH,
                        )
                acc = jnp.dot(buf_l2[pl.ds(off, rows), :], w_vmem[...],
                              preferred_element_type=jnp.int32)
                outs.append(emit(3, off, opp * m_per + off, rows,
                                 acc.astype(jnp.float32) * scale, 3 + 2 * j))
                if not last:
                    fwd_l[j].wait_recv()
                acc = jnp.dot(buf_r2[pl.ds(off, rows), :], w_vmem[...],
                              preferred_element_type=jnp.int32)
                outs.append(emit(4, off, opp * m_per + half + off, rows,
                                 acc.astype(jnp.float32) * scale, 4 + 2 * j))

            for cp in outs:
                cp.wait()
            for s in [send_r1, send_l1, send_r2, send_l2] + fwd_r + fwd_l:
                s.wait_send()

            pl.semaphore_wait(second_barrier, 2)

    return pl.pallas_call(
        body,
        out_shape=jax.ShapeDtypeStruct((N_DEV * m_per, n_per), jnp.float32),
        in_specs=[
            pl.BlockSpec(memory_space=pl.ANY),
            pl.BlockSpec(memory_space=pl.ANY),
            pl.BlockSpec(memory_space=pltpu.VMEM),
            pl.BlockSpec(memory_space=pltpu.VMEM),
        ],
        out_specs=pl.BlockSpec(memory_space=pl.ANY),
        scratch_shapes=[
            pltpu.VMEM((m_per, k), x.dtype),
            pltpu.VMEM((k, n_per), w_mat.dtype),
            pltpu.VMEM((m_per, k), x.dtype),
            pltpu.VMEM((m_per, k), x.dtype),
            pltpu.VMEM((half, k), x.dtype),
            pltpu.VMEM((half, k), x.dtype),
            pltpu.VMEM((5, m_per, n_per), jnp.float32),
            pltpu.SemaphoreType.DMA((2,)),
            pltpu.SemaphoreType.DMA((9,)),
            pltpu.SemaphoreType.DMA((10,)),
            pltpu.SemaphoreType.DMA((10,)),
        ],
        compiler_params=pltpu.CompilerParams(collective_id=0),
    )(x, w_mat, scale_x, scale_w)
